# baseline (speedup 1.0000x reference)
import functools

import jax
import jax.numpy as jnp
from jax import lax
from jax.experimental import pallas as pl
from jax.experimental.pallas import tpu as pltpu
from jax.experimental.pallas import tpu_sc as plsc

_DIM2 = 32
_BATCH = 16384
_FIELDS = 26
_B = _BATCH * _FIELDS

_info = plsc.get_sparse_core_info()
_NC = _info.num_cores
_NS = _info.num_subcores
_NW = _NC * _NS
_B_PER_W = _B // _NW
_BW = _BATCH // _NW

_CH = 104
_NCH = _B_PER_W // _CH
_SUBB = 32
_SUB = _SUBB * _FIELDS
_CPS = _SUB // _CH
_NSUB = _BW // _SUBB
_NP = _NSUB // 2


@functools.partial(
    pl.kernel,
    mesh=plsc.VectorSubcoreMesh(core_axis_name="c", subcore_axis_name="s"),
    out_type=jax.ShapeDtypeStruct((_FIELDS, _DIM2, _BATCH), jnp.float32),
    scratch_types=[
        pltpu.VMEM((_NCH, _CH), jnp.int32),
        pltpu.VMEM((2 * _SUB, _DIM2), jnp.float32),
        pltpu.VMEM((_FIELDS, _DIM2, _SUBB), jnp.float32),
        pltpu.VMEM((_FIELDS, _DIM2, _SUBB), jnp.float32),
        pltpu.SemaphoreType.DMA,
    ],
    compiler_params=pltpu.CompilerParams(
        use_tc_tiling_on_sc=False, needs_layout_passes=False),
)
def _gather_t(x_hbm, table_hbm, out_hbm, idx_v, rbuf, tb0, tb1, gsem):
    wid = lax.axis_index("s") * _NC + lax.axis_index("c")
    pltpu.sync_copy(x_hbm.at[wid], idx_v)
    iota = lax.iota(jnp.int32, 16)
    row_base = iota * _FIELDS

    def fire(s, half):
        return [
            pltpu.async_copy(
                table_hbm.at[idx_v.at[s * _CPS + k]],
                rbuf.at[pl.ds(half * _SUB + k * _CH, _CH)],
                gsem)
            for k in range(_CPS)
        ]

    def transpose(half, tb):
        src_off = half * _SUB

        @pl.loop(0, _FIELDS)
        def _f(f):
            for h in (0, 1):
                rowv = row_base + (src_off + 416 * h + f)
                fv = jnp.full((16,), 0, jnp.int32) + f
                bv = iota + 16 * h
                for c in range(_DIM2):
                    v = plsc.load_gather(
                        rbuf, [rowv, jnp.full((16,), c, jnp.int32)])
                    plsc.store_scatter(
                        tb, [fv, jnp.full((16,), c, jnp.int32), bv], v)

    @pl.loop(0, _NP)
    def _pairs(p):
        s0 = 2 * p
        h0 = fire(s0, 0)
        h1 = fire(s0 + 1, 1)
        for h in h0:
            h.wait()
        transpose(0, tb0)
        b0 = wid * _BW + s0 * _SUBB
        pltpu.sync_copy(tb0, out_hbm.at[:, :, pl.ds(b0, _SUBB)])
        for h in h1:
            h.wait()
        transpose(1, tb1)
        pltpu.sync_copy(tb1, out_hbm.at[:, :, pl.ds(b0 + _SUBB, _SUBB)])


def kernel(x, table):
    xw = x.reshape(_NW, _NCH, _CH)
    out_t = _gather_t(xw, table)
    return out_t.transpose(2, 0, 1)


# speedup vs baseline: 1.0745x; 1.0745x over previous
import functools

import jax
import jax.numpy as jnp
from jax import lax
from jax.experimental import pallas as pl
from jax.experimental.pallas import tpu as pltpu
from jax.experimental.pallas import tpu_sc as plsc

_DIM2 = 32
_BATCH = 16384
_FIELDS = 26
_B = _BATCH * _FIELDS

_info = plsc.get_sparse_core_info()
_NC = _info.num_cores
_NS = _info.num_subcores
_NW = _NC * _NS
_B_PER_W = _B // _NW
_BW = _BATCH // _NW

_CH = 104
_NCH = _B_PER_W // _CH
_SUBB = 32
_SUB = _SUBB * _FIELDS
_CPS = _SUB // _CH
_NSUB = _BW // _SUBB
_NP = _NSUB // 2


@functools.partial(
    pl.kernel,
    mesh=plsc.VectorSubcoreMesh(core_axis_name="c", subcore_axis_name="s"),
    out_type=jax.ShapeDtypeStruct((_FIELDS, _DIM2, _BATCH), jnp.float32),
    scratch_types=[
        pltpu.VMEM((_NCH, _CH), jnp.int32),
        pltpu.VMEM((2 * _SUB, _DIM2), jnp.float32),
        pltpu.VMEM((_FIELDS, _DIM2, _SUBB), jnp.float32),
        pltpu.VMEM((_FIELDS, _DIM2, _SUBB), jnp.float32),
        pltpu.SemaphoreType.DMA,
    ],
    compiler_params=pltpu.CompilerParams(
        use_tc_tiling_on_sc=False, needs_layout_passes=False),
)
def _gather_t(x_hbm, table_hbm, out_hbm, idx_v, rbuf, tb0, tb1, gsem):
    wid = lax.axis_index("s") * _NC + lax.axis_index("c")
    pltpu.sync_copy(x_hbm.at[wid], idx_v)
    iota = lax.iota(jnp.int32, 16)
    row_base = iota * _FIELDS

    def fire(s, half):
        return [
            pltpu.async_copy(
                table_hbm.at[idx_v.at[s * _CPS + k]],
                rbuf.at[pl.ds(half * _SUB + k * _CH, _CH)],
                gsem)
            for k in range(_CPS)
        ]

    fvs = [jnp.full((16,), f, jnp.int32) for f in range(_FIELDS)]
    cvs = (iota, iota + 16)

    def transpose(half, tb):
        src_off = half * _SUB

        @pl.loop(0, _SUBB)
        def _b(b_l):
            row0 = b_l * _FIELDS + src_off
            bv = jnp.full((16,), 0, jnp.int32) + b_l
            for f in range(_FIELDS):
                for h in (0, 1):
                    v = rbuf[row0 + f, pl.ds(16 * h, 16)]
                    plsc.store_scatter(tb, [fvs[f], cvs[h], bv], v)

    @pl.loop(0, _NP)
    def _pairs(p):
        s0 = 2 * p
        h0 = fire(s0, 0)
        h1 = fire(s0 + 1, 1)
        for h in h0:
            h.wait()
        transpose(0, tb0)
        b0 = wid * _BW + s0 * _SUBB
        pltpu.sync_copy(tb0, out_hbm.at[:, :, pl.ds(b0, _SUBB)])
        for h in h1:
            h.wait()
        transpose(1, tb1)
        pltpu.sync_copy(tb1, out_hbm.at[:, :, pl.ds(b0 + _SUBB, _SUBB)])


def kernel(x, table):
    xw = x.reshape(_NW, _NCH, _CH)
    out_t = _gather_t(xw, table)
    return out_t.transpose(2, 0, 1)
